# baseline (speedup 1.0000x reference)
import jax, jax.numpy as jnp
from jax.experimental import pallas as pl

_NPOINT = 512
_RADIUS_LIST = [0.1, 0.2, 0.4]
_NSAMPLE_LIST = [16, 32, 128]


def _square_distance(src, dst):
    d = -2.0 * jnp.einsum('bsc,bnc->bsn', src, dst)
    d = d + jnp.sum(src ** 2, axis=-1)[:, :, None]
    d = d + jnp.sum(dst ** 2, axis=-1)[:, None, :]
    return d


def _fps(xyz, npoint):
    _, n, _ = xyz.shape

    def one(pts):
        def body(i, state):
            dists, idxs, last_idx = state
            last = pts[last_idx]
            d = jnp.sum((pts - last[None, :]) ** 2, axis=-1)
            dists = jnp.minimum(dists, d)
            nxt = jnp.argmax(dists).astype(jnp.int32)
            idxs = idxs.at[i].set(nxt)
            return (dists, idxs, nxt)

        dists0 = jnp.full((n,), 1e10, dtype=jnp.float32)
        idxs0 = jnp.zeros((npoint,), dtype=jnp.int32)
        _, idxs, _ = jax.lax.fori_loop(1, npoint, body, (dists0, idxs0, jnp.int32(0)))
        return idxs

    return jax.vmap(one)(xyz)


def _index_points(points, idx):
    return jax.vmap(lambda p, i: p[i])(points, idx)


def _query_ball(radius, nsample, xyz, new_xyz):
    b, n, _ = xyz.shape
    s = new_xyz.shape[1]
    sqrdists = _square_distance(new_xyz, xyz)
    group_idx = jnp.broadcast_to(jnp.arange(n, dtype=jnp.int32), (b, s, n))
    group_idx = jnp.where(sqrdists > radius ** 2, jnp.int32(n), group_idx)
    group_idx = jnp.sort(group_idx, axis=-1)[:, :, :nsample]
    group_first = group_idx[:, :, 0:1]
    group_idx = jnp.where(group_idx == n, jnp.broadcast_to(group_first, group_idx.shape), group_idx)
    return group_idx


def kernel(xyz, points, params):
    xyz_t = jnp.transpose(xyz, (0, 2, 1))
    pts_t = jnp.transpose(points, (0, 2, 1))
    S = _NPOINT
    fps_idx = _fps(jax.lax.stop_gradient(xyz_t), S)
    new_xyz = _index_points(xyz_t, fps_idx)
    new_points_list = []
    for i, radius in enumerate(_RADIUS_LIST):
        K = _NSAMPLE_LIST[i]
        group_idx = _query_ball(radius, K, xyz_t, new_xyz)
        grouped_xyz = _index_points(xyz_t, group_idx) - new_xyz[:, :, None, :]
        grouped_points = _index_points(pts_t, group_idx)
        grouped_points = jnp.concatenate([grouped_points, grouped_xyz], axis=-1)
        g = jnp.transpose(grouped_points, (0, 3, 2, 1))
        for layer in params[i]:
            g = jnp.einsum('oc,bcks->boks', layer['W'], g) + layer['b'][None, :, None, None]
            mean = jnp.mean(g, axis=(0, 2, 3), keepdims=True)
            var = jnp.var(g, axis=(0, 2, 3), keepdims=True)
            g = (g - mean) / jnp.sqrt(var + 1e-5)
            g = g * layer['gamma'][None, :, None, None] + layer['beta'][None, :, None, None]
            g = jax.nn.relu(g)
        new_points_list.append(jnp.max(g, axis=2))
    return (jnp.transpose(new_xyz, (0, 2, 1)), jnp.concatenate(new_points_list, axis=1))


# speedup vs baseline: 13.4796x; 13.4796x over previous
import functools

import jax
import jax.numpy as jnp
from jax import lax
from jax.experimental import pallas as pl
from jax.experimental.pallas import tpu as pltpu
from jax.experimental.pallas import tpu_sc as plsc

_B, _N = 8, 4096
_S = 512
_RADII = (0.1, 0.2, 0.4)
_KS = (16, 32, 128)
_F32 = jnp.float32
_I32 = jnp.int32


def _fps_body(xc_ref, yc_ref, zc_ref, ox_ref, oy_ref, oz_ref):
    xc = xc_ref[...]
    yc = yc_ref[...]
    zc = zc_ref[...]
    lane = lax.broadcasted_iota(_I32, (_B, _N), 1)

    lx = xc[:, 0:1]
    ly = yc[:, 0:1]
    lz = zc[:, 0:1]
    ox_ref[0:1, :] = jnp.reshape(lx, (1, _B))
    oy_ref[0:1, :] = jnp.reshape(ly, (1, _B))
    oz_ref[0:1, :] = jnp.reshape(lz, (1, _B))

    def body(i, st):
        dists, lx, ly, lz = st
        dx = xc - lx
        dy = yc - ly
        dz = zc - lz
        d = dx * dx + dy * dy + dz * dz
        dists = jnp.minimum(dists, d)
        nxt = jnp.argmax(dists, axis=1).astype(_I32)
        oh = lane == nxt[:, None]
        lx = jnp.sum(jnp.where(oh, xc, 0.0), axis=1, keepdims=True)
        ly = jnp.sum(jnp.where(oh, yc, 0.0), axis=1, keepdims=True)
        lz = jnp.sum(jnp.where(oh, zc, 0.0), axis=1, keepdims=True)
        ox_ref[pl.ds(i, 1), :] = jnp.reshape(lx, (1, _B))
        oy_ref[pl.ds(i, 1), :] = jnp.reshape(ly, (1, _B))
        oz_ref[pl.ds(i, 1), :] = jnp.reshape(lz, (1, _B))
        return dists, lx, ly, lz

    dists0 = jnp.full((_B, _N), 1e10, dtype=_F32)
    lax.fori_loop(1, _S, body, (dists0, lx, ly, lz))


def _run_fps(xyz):
    out = pl.pallas_call(
        _fps_body,
        grid=(1,),
        in_specs=[pl.BlockSpec((_B, _N), lambda i: (0, 0))] * 3,
        out_specs=[pl.BlockSpec((_S, _B), lambda i: (0, 0))] * 3,
        out_shape=[jax.ShapeDtypeStruct((_S, _B), _F32)] * 3,
    )(xyz[:, 0, :], xyz[:, 1, :], xyz[:, 2, :])
    return out


_ST = 256


def _dist_body(newc_ref, xyzp_ref, out_ref):
    X = xyzp_ref[0]
    C = newc_ref[0]
    Ct = jnp.transpose(C, (1, 0))
    dot = lax.dot_general(Ct, X, (((1,), (0,)), ((), ())),
                          preferred_element_type=_F32)
    snorm = jnp.sum(Ct * Ct, axis=1, keepdims=True)
    nnorm = jnp.sum(X * X, axis=0, keepdims=True)
    out_ref[0] = snorm + nnorm - 2.0 * dot


def _run_dist(newc, xyzp):
    return pl.pallas_call(
        _dist_body,
        grid=(_B, _S // _ST),
        in_specs=[
            pl.BlockSpec((1, 8, _ST), lambda b, s: (b, 0, s)),
            pl.BlockSpec((1, 8, _N), lambda b, s: (b, 0, 0)),
        ],
        out_specs=pl.BlockSpec((1, _ST, _N), lambda b, s: (b, s, 0)),
        out_shape=jax.ShapeDtypeStruct((_B, _S, _N), _F32),
    )(newc, xyzp)


_NW = 32
_RPW = (_B * _S) // _NW
_NCH = _N // 16


_RB = 8


def _sc_body(dist_hbm, feat_hbm, o1, o2, o3,
             drows, idx1, idx2, idx3, gi1, gi2, gi3,
             rows1, rows2, rows3, dsem, gsem1, gsem2, gsem3):
    wid = lax.axis_index("s") * 2 + lax.axis_index("c")
    r1s = jnp.float32(_RADII[0] ** 2)
    r2s = jnp.float32(_RADII[1] ** 2)
    r3s = jnp.float32(_RADII[2] ** 2)
    lane16 = lax.iota(_I32, 16)

    def do_batch(g, carry):
        row0 = wid * _RPW + g * _RB
        pltpu.sync_copy(dist_hbm.at[pl.ds(row0, _RB)], drows)

        def do_row(rr, carry2):
            row = row0 + rr
            b = row // _S
            base = b * _N
            drow = drows.at[rr]

            def condA(st):
                c, f1, f2, f3 = st
                return jnp.logical_and(
                    c < _NCH, jnp.logical_or(f2 < _KS[1], f3 < _KS[2]))

            def stepA(st):
                c, f1, f2, f3 = st
                d = drow[pl.ds(c * 16, 16)]
                vals = lane16 + (c * 16 + base)
                m1 = d <= r1s
                m2 = d <= r2s
                m3 = d <= r3s
                plsc.store_compressed(idx1.at[pl.ds(f1, 16)], vals, mask=m1)
                plsc.store_compressed(idx2.at[pl.ds(f2, 16)], vals, mask=m2)
                plsc.store_compressed(idx3.at[pl.ds(f3, 16)], vals, mask=m3)
                f1 = jnp.minimum(f1 + jnp.sum(m1.astype(_I32)), _KS[0])
                f2 = jnp.minimum(f2 + jnp.sum(m2.astype(_I32)), _KS[1])
                f3 = jnp.minimum(f3 + jnp.sum(m3.astype(_I32)), _KS[2])
                return c + 1, f1, f2, f3

            c0, f1, f2, f3 = lax.while_loop(
                condA, stepA,
                (jnp.int32(0), jnp.int32(0), jnp.int32(0), jnp.int32(0)))

            def condB(st):
                c, f1 = st
                return jnp.logical_and(c < _NCH, f1 < _KS[0])

            def stepB(st):
                c, f1 = st
                d = drow[pl.ds(c * 16, 16)]
                m1 = d <= r1s
                plsc.store_compressed(idx1.at[pl.ds(f1, 16)],
                                      lane16 + (c * 16 + base), mask=m1)
                return c + 1, jnp.minimum(f1 + jnp.sum(m1.astype(_I32)), _KS[0])

            _, f1 = lax.while_loop(condB, stepB, (c0, f1))

            for idxb, gib, K, fcnt in ((idx1, gi1, _KS[0], f1),
                                       (idx2, gi2, _KS[1], f2),
                                       (idx3, gi3, _KS[2], f3)):
                first = jnp.full((16,), idxb[pl.ds(0, 16)][0], _I32)
                for j in range(K // 16):
                    cur = idxb[pl.ds(j * 16, 16)]
                    slot = lane16 + (j * 16)
                    gib[pl.ds(j * 16, 16)] = jnp.where(slot < fcnt, cur, first)

            cp1 = pltpu.async_copy(feat_hbm.at[gi1], rows1, gsem1)
            cp2 = pltpu.async_copy(feat_hbm.at[gi2], rows2, gsem2)
            cp3 = pltpu.async_copy(feat_hbm.at[gi3], rows3, gsem3)
            cp1.wait()
            pltpu.sync_copy(rows1, o1.at[pl.ds(row * _KS[0], _KS[0])])
            cp2.wait()
            pltpu.sync_copy(rows2, o2.at[pl.ds(row * _KS[1], _KS[1])])
            cp3.wait()
            pltpu.sync_copy(rows3, o3.at[pl.ds(row * _KS[2], _KS[2])])
            return carry2

        lax.fori_loop(0, _RB, do_row, jnp.int32(0))
        return carry

    lax.fori_loop(0, _RPW // _RB, do_batch, jnp.int32(0))


def _run_ballquery(dist2d, feat):
    mesh = plsc.VectorSubcoreMesh(core_axis_name="c", subcore_axis_name="s")
    kfn = pl.kernel(
        _sc_body,
        out_type=[
            jax.ShapeDtypeStruct((_B * _S * _KS[0], 16), _F32),
            jax.ShapeDtypeStruct((_B * _S * _KS[1], 16), _F32),
            jax.ShapeDtypeStruct((_B * _S * _KS[2], 16), _F32),
        ],
        mesh=mesh,
        scratch_types=[
            pltpu.VMEM((_RB, _N), _F32),
            pltpu.VMEM((_KS[0] + 16,), _I32),
            pltpu.VMEM((_KS[1] + 16,), _I32),
            pltpu.VMEM((_KS[2] + 16,), _I32),
            pltpu.VMEM((_KS[0],), _I32),
            pltpu.VMEM((_KS[1],), _I32),
            pltpu.VMEM((_KS[2],), _I32),
            pltpu.VMEM((_KS[0], 16), _F32),
            pltpu.VMEM((_KS[1], 16), _F32),
            pltpu.VMEM((_KS[2], 16), _F32),
            pltpu.SemaphoreType.DMA,
            pltpu.SemaphoreType.DMA,
            pltpu.SemaphoreType.DMA,
            pltpu.SemaphoreType.DMA,
        ],
        compiler_params=pltpu.CompilerParams(
            needs_layout_passes=False, use_tc_tiling_on_sc=False),
    )
    return kfn(dist2d, feat)


def _mlp_layer_body(x_ref, w_ref, b_ref, sc_ref, sh_ref, y_ref, st_ref,
                    *, first, gt, k, cin, cout, count):
    if first:
        c = sc_ref[...]
        x = jnp.reshape(x_ref[...] - c[:, None, :], (gt * k, cin))
    else:
        st = sc_ref[...]
        s1 = jnp.sum(st[:, 0, :], axis=0)
        s2 = jnp.sum(st[:, 1, :], axis=0)
        mean = s1 * (1.0 / count)
        var = s2 * (1.0 / count) - mean * mean
        scale = sh_ref[0, :] / jnp.sqrt(var + 1e-5)
        shift = sh_ref[1, :] - mean * scale
        x = jnp.reshape(x_ref[...], (gt * k, cin))
        x = jnp.maximum(x * scale[None, :] + shift[None, :], 0.0)
    w = w_ref[...]
    y = lax.dot_general(x, w, (((1,), (0,)), ((), ())),
                        preferred_element_type=_F32)
    y = y + b_ref[0:1, :]
    y_ref[...] = jnp.reshape(y, (gt, k, cout))
    s1 = jnp.sum(y, axis=0)
    s2 = jnp.sum(y * y, axis=0)
    st_ref[0] = jnp.concatenate(
        [s1[None, :], s2[None, :], jnp.zeros((6, cout), _F32)], axis=0)


def _run_mlp_layer(x, w, bias, scstat, gb, *, first, gt, k, cin, cout):
    g = (_B * _S) // gt
    count = float(_B * _S * k)
    body = functools.partial(_mlp_layer_body, first=first, gt=gt, k=k,
                             cin=cin, cout=cout, count=count)
    if first:
        sc_spec = pl.BlockSpec((gt, cin), lambda i: (i, 0))
    else:
        sc_spec = pl.BlockSpec((g, 8, cin), lambda i: (0, 0, 0))
    y, st = pl.pallas_call(
        body,
        grid=(g,),
        in_specs=[
            pl.BlockSpec((gt, k, cin), lambda i: (i, 0, 0)),
            pl.BlockSpec((cin, cout), lambda i: (0, 0)),
            pl.BlockSpec((8, cout), lambda i: (0, 0)),
            sc_spec,
            pl.BlockSpec((8, cin), lambda i: (0, 0)),
        ],
        out_specs=[
            pl.BlockSpec((gt, k, cout), lambda i: (i, 0, 0)),
            pl.BlockSpec((1, 8, cout), lambda i: (i, 0, 0)),
        ],
        out_shape=[
            jax.ShapeDtypeStruct((_B * _S, k, cout), _F32),
            jax.ShapeDtypeStruct((g, 8, cout), _F32),
        ],
    )(x, w, bias, scstat, gb)
    return y, st


def _mlp_final_body(x_ref, st_ref, gb_ref, o_ref, *, gt, k, c, count):
    st = st_ref[...]
    s1 = jnp.sum(st[:, 0, :], axis=0)
    s2 = jnp.sum(st[:, 1, :], axis=0)
    mean = s1 * (1.0 / count)
    var = s2 * (1.0 / count) - mean * mean
    scale = gb_ref[0, :] / jnp.sqrt(var + 1e-5)
    shift = gb_ref[1, :] - mean * scale
    x = x_ref[...]
    x = jnp.maximum(x * scale[None, None, :] + shift[None, None, :], 0.0)
    o_ref[...] = jnp.max(x, axis=1)


def _run_mlp_final(y, st, gb, *, gt, k, c):
    g = (_B * _S) // gt
    count = float(_B * _S * k)
    body = functools.partial(_mlp_final_body, gt=gt, k=k, c=c, count=count)
    return pl.pallas_call(
        body,
        grid=(g,),
        in_specs=[
            pl.BlockSpec((gt, k, c), lambda i: (i, 0, 0)),
            pl.BlockSpec((g, 8, c), lambda i: (0, 0, 0)),
            pl.BlockSpec((8, c), lambda i: (0, 0)),
        ],
        out_specs=pl.BlockSpec((gt, c), lambda i: (i, 0)),
        out_shape=jax.ShapeDtypeStruct((_B * _S, c), _F32),
    )(y, st, gb)


def kernel(xyz, points, params):
    ox, oy, oz = _run_fps(xyz)

    xyzp = jnp.concatenate(
        [xyz, jnp.zeros((_B, 5, _N), _F32)], axis=1)
    newc = jnp.concatenate(
        [jnp.stack([ox.T, oy.T, oz.T], axis=1),
         jnp.zeros((_B, 5, _S), _F32)], axis=1)

    dist = _run_dist(newc, xyzp)
    dist2d = jnp.reshape(dist, (_B * _S, _N))

    feat = jnp.concatenate(
        [jnp.transpose(points, (0, 2, 1)),
         jnp.transpose(xyz, (0, 2, 1)),
         jnp.zeros((_B, _N, 10), _F32)], axis=2)
    feat = jnp.reshape(feat, (_B * _N, 16))

    g1, g2, g3 = _run_ballquery(dist2d, feat)

    cext = jnp.concatenate(
        [jnp.zeros((_B * _S, 3), _F32),
         jnp.reshape(jnp.stack([ox.T, oy.T, oz.T], axis=2), (_B * _S, 3)),
         jnp.zeros((_B * _S, 10), _F32)], axis=1)

    gts = (64, 32, 16)
    outs = []
    for gi, (grp, K, br) in enumerate(((g1, _KS[0], params[0]),
                                       (g2, _KS[1], params[1]),
                                       (g3, _KS[2], params[2]))):
        gt = gts[gi]
        g = (_B * _S) // gt
        x = jnp.reshape(grp, (_B * _S, K, 16))
        cin = 16
        scstat, gb = cext, jnp.zeros((8, 16), _F32)
        for li, layer in enumerate(br):
            cout = layer['W'].shape[0]
            wp = jnp.zeros((cin, cout), _F32)
            wp = wp.at[:layer['W'].shape[1], :].set(layer['W'].T)
            bias = jnp.broadcast_to(layer['b'][None, :], (8, cout))
            y, st = _run_mlp_layer(x, wp, bias, scstat, gb,
                                   first=(li == 0), gt=gt, k=K,
                                   cin=cin, cout=cout)
            gb = jnp.concatenate(
                [layer['gamma'][None, :], layer['beta'][None, :],
                 jnp.zeros((6, cout), _F32)], axis=0)
            scstat = st
            x, cin = y, cout
        o = _run_mlp_final(x, scstat, gb, gt=gt, k=K, c=cin)
        outs.append(jnp.transpose(jnp.reshape(o, (_B, _S, cin)), (0, 2, 1)))

    new_xyz_out = jnp.stack([ox.T, oy.T, oz.T], axis=1)
    return (new_xyz_out, jnp.concatenate(outs, axis=1))
